# baseline (speedup 1.0000x reference)
import jax
import jax.numpy as jnp
from jax.experimental import pallas as pl
from jax.experimental.pallas import tpu as pltpu

TOPK = 8
BT = 1024


def _router_block(x_ref, wt_ref, w_out_ref, i_out_ref):
    s = jnp.dot(x_ref[...], wt_ref[...], preferred_element_type=jnp.float32)
    m = jnp.max(s, axis=-1, keepdims=True)
    z = jnp.sum(jnp.exp(s - m), axis=-1, keepdims=True)

    bits = jax.lax.bitcast_convert_type(s, jnp.int32)
    key = bits ^ (jax.lax.shift_right_arithmetic(bits, 31) & jnp.int32(0x7FFFFFFF))
    col = jax.lax.broadcasted_iota(jnp.int32, s.shape, 1)
    packed = (key & jnp.int32(~63)) | (jnp.int32(63) - col)

    keys = []
    idxs = []
    for _ in range(TOPK):
        pk = jnp.max(packed, axis=-1, keepdims=True)
        idx = jnp.int32(63) - (pk & jnp.int32(63))
        keys.append(pk)
        idxs.append(idx)
        packed = jnp.where(col == idx, jnp.int32(-0x80000000), packed)

    pk8 = jnp.concatenate(keys, axis=-1)
    k8 = pk8 & jnp.int32(~63)
    b8 = k8 ^ (jax.lax.shift_right_arithmetic(k8, 31) & jnp.int32(0x7FFFFFFF))
    s8 = jax.lax.bitcast_convert_type(b8, jnp.float32)
    w_out_ref[...] = jnp.exp(s8 - m) / z
    i_out_ref[...] = jnp.concatenate(idxs, axis=-1)


@jax.jit
def kernel(x, W):
    T, D = x.shape
    E = W.shape[0]
    wt = W.T
    grid = (T // BT,)
    weights, indices = pl.pallas_call(
        _router_block,
        grid=grid,
        in_specs=[
            pl.BlockSpec((BT, D), lambda i: (i, 0)),
            pl.BlockSpec((D, E), lambda i: (0, 0)),
        ],
        out_specs=[
            pl.BlockSpec((BT, TOPK), lambda i: (i, 0)),
            pl.BlockSpec((BT, TOPK), lambda i: (i, 0)),
        ],
        out_shape=[
            jax.ShapeDtypeStruct((T, TOPK), jnp.float32),
            jax.ShapeDtypeStruct((T, TOPK), jnp.int32),
        ],
        compiler_params=pltpu.CompilerParams(
            dimension_semantics=("arbitrary",),
        ),
    )(x, wt)
    return weights, indices


# speedup vs baseline: 1.0701x; 1.0701x over previous
import jax
import jax.numpy as jnp
from jax.experimental import pallas as pl
from jax.experimental.pallas import tpu as pltpu

TOPK = 8
BT = 1024


def _router_block(x_ref, wt_ref, w_out_ref, i_out_ref):
    s = jnp.dot(x_ref[...], wt_ref[...], preferred_element_type=jnp.float32)
    m = jnp.max(s, axis=-1, keepdims=True)
    z = jnp.sum(jnp.exp(s - m), axis=-1, keepdims=True)

    col = jax.lax.broadcasted_iota(jnp.int32, s.shape, 1)
    colf = col.astype(jnp.float32)
    bits = jax.lax.bitcast_convert_type(s, jnp.int32)
    packed = jax.lax.bitcast_convert_type(
        (bits & jnp.int32(~63)) | (jnp.int32(63) - col), jnp.float32)

    svals = []
    idxs = []
    for _ in range(TOPK):
        pk = jnp.max(packed, axis=-1, keepdims=True)
        pkb = jax.lax.bitcast_convert_type(pk, jnp.int32)
        idx = jnp.int32(63) - (pkb & jnp.int32(63))
        svals.append(jax.lax.bitcast_convert_type(pkb & jnp.int32(~63),
                                                  jnp.float32))
        idxs.append(idx)
        packed = jnp.where(colf == idx.astype(jnp.float32), -jnp.inf, packed)

    s8 = jnp.concatenate(svals, axis=-1)
    w_out_ref[...] = jnp.exp(s8 - m) / z
    i_out_ref[...] = jnp.concatenate(idxs, axis=-1)


@jax.jit
def kernel(x, W):
    T, D = x.shape
    E = W.shape[0]
    wt = W.T
    grid = (T // BT,)
    weights, indices = pl.pallas_call(
        _router_block,
        grid=grid,
        in_specs=[
            pl.BlockSpec((BT, D), lambda i: (i, 0)),
            pl.BlockSpec((D, E), lambda i: (0, 0)),
        ],
        out_specs=[
            pl.BlockSpec((BT, TOPK), lambda i: (i, 0)),
            pl.BlockSpec((BT, TOPK), lambda i: (i, 0)),
        ],
        out_shape=[
            jax.ShapeDtypeStruct((T, TOPK), jnp.float32),
            jax.ShapeDtypeStruct((T, TOPK), jnp.int32),
        ],
        compiler_params=pltpu.CompilerParams(
            dimension_semantics=("arbitrary",),
        ),
    )(x, wt)
    return weights, indices
